# baseline (speedup 1.0000x reference)
import jax
import jax.numpy as jnp
from jax import lax
from jax.experimental import pallas as pl


def _argmax_body(x_ref, o_ref):
    blk = x_ref[0]
    m = blk.shape[0]
    mx = jnp.max(blk, axis=0)
    iota = lax.broadcasted_iota(jnp.int32, blk.shape, 0)
    idx = jnp.min(jnp.where(blk == mx[None, :], iota, m), axis=0)
    o_ref[0, 0] = idx


def kernel(x):
    B, M, N = x.shape
    NB = 128
    out = pl.pallas_call(
        _argmax_body,
        grid=(B, N // NB),
        in_specs=[pl.BlockSpec((1, M, NB), lambda b, n: (b, 0, n))],
        out_specs=pl.BlockSpec((1, 1, NB), lambda b, n: (b, 0, n)),
        out_shape=jax.ShapeDtypeStruct((B, 1, N), jnp.int32),
    )(x)
    return out.reshape(B, N)


# speedup vs baseline: 1.8716x; 1.8716x over previous
import jax
import jax.numpy as jnp
from jax import lax
from jax.experimental import pallas as pl


def _argmax_body(x_ref, o_ref):
    blk = x_ref[0]
    m = blk.shape[0]
    mx = jnp.max(blk, axis=0)
    iota = lax.broadcasted_iota(jnp.int32, blk.shape, 0)
    idx = jnp.min(jnp.where(blk == mx[None, :], iota, m), axis=0)
    o_ref[0, 0] = idx


def kernel(x):
    B, M, N = x.shape
    NB = 512
    out = pl.pallas_call(
        _argmax_body,
        grid=(B, N // NB),
        in_specs=[pl.BlockSpec((1, M, NB), lambda b, n: (b, 0, n))],
        out_specs=pl.BlockSpec((1, 1, NB), lambda b, n: (b, 0, n)),
        out_shape=jax.ShapeDtypeStruct((B, 1, N), jnp.int32),
    )(x)
    return out.reshape(B, N)


# speedup vs baseline: 1.9512x; 1.0425x over previous
import jax
import jax.numpy as jnp
from jax import lax
from jax.experimental import pallas as pl


def _argmax_body(x_ref, o_ref):
    blk = x_ref[0]
    m = blk.shape[0]
    mx = jnp.max(blk, axis=0)
    iota = lax.broadcasted_iota(jnp.int32, blk.shape, 0)
    idx = jnp.min(jnp.where(blk == mx[None, :], iota, m), axis=0)
    o_ref[0, 0] = idx


def kernel(x):
    B, M, N = x.shape
    NB = 1024
    out = pl.pallas_call(
        _argmax_body,
        grid=(B, N // NB),
        in_specs=[pl.BlockSpec((1, M, NB), lambda b, n: (b, 0, n))],
        out_specs=pl.BlockSpec((1, 1, NB), lambda b, n: (b, 0, n)),
        out_shape=jax.ShapeDtypeStruct((B, 1, N), jnp.int32),
    )(x)
    return out.reshape(B, N)
